# baseline (speedup 1.0000x reference)
import functools

import jax
import jax.numpy as jnp
from jax import lax
from jax.experimental import pallas as pl
from jax.experimental.pallas import tpu as pltpu
from jax.experimental.pallas import tpu_sc as plsc

_N_TOKEN = 1000000
_D = 128
_CUT = (0, 20000, 100000, 500000, 1000000)
_HALF = 500000
_SCALE = float(_D) ** 0.5

_BLK = 4000

_NC = 2
_NS = 16
_NW = _NC * _NS
_B_TOTAL = 4096 * 200
_BPW = _B_TOTAL // _NW
_CH = 64
_NCHUNK = _BPW // _CH
_NBUF = 4
_NROUND = _NCHUNK // _NBUF


def _pack_bf16_pair(dot_e, dot_o):
    e16 = jax.lax.bitcast_convert_type(dot_e.astype(jnp.bfloat16), jnp.int16)
    o16 = jax.lax.bitcast_convert_type(dot_o.astype(jnp.bfloat16), jnp.int16)
    return (o16.astype(jnp.int32) << 16) | (e16.astype(jnp.int32) & 0xFFFF)


def _proj_body(emb_ref, pje_ref, pjo_ref, e3_ref, p3e_ref, p3o_ref,
               u_ref, out_ref):
    del u_ref
    lo = _pack_bf16_pair(
        jnp.dot(emb_ref[...], pje_ref[...],
                preferred_element_type=jnp.float32) * _SCALE,
        jnp.dot(emb_ref[...], pjo_ref[...],
                preferred_element_type=jnp.float32) * _SCALE,
    )
    hi = _pack_bf16_pair(
        jnp.dot(e3_ref[...], p3e_ref[...],
                preferred_element_type=jnp.float32) * _SCALE,
        jnp.dot(e3_ref[...], p3o_ref[...],
                preferred_element_type=jnp.float32) * _SCALE,
    )
    out_ref[...] = jnp.concatenate([lo, hi], axis=1)


def _project_cluster(emb, pje, pjo, emb3, p3e, p3o, unified, row_off):
    rows, d = emb.shape
    grid = (rows // _BLK,)
    off_blocks = row_off // _BLK
    in_specs = [
        pl.BlockSpec((_BLK, d), lambda b: (b, 0)),
        pl.BlockSpec((d, 64), lambda b: (0, 0)),
        pl.BlockSpec((d, 64), lambda b: (0, 0)),
        pl.BlockSpec((_BLK, 2), lambda b, _o=off_blocks: (b + _o, 0)),
        pl.BlockSpec((2, 64), lambda b: (0, 0)),
        pl.BlockSpec((2, 64), lambda b: (0, 0)),
    ]
    out_spec = pl.BlockSpec((_BLK, _D), lambda b, _o=off_blocks: (b + _o, 0))
    out_shape = jax.ShapeDtypeStruct((_HALF, _D), jnp.int32)
    if unified is None:
        return pl.pallas_call(
            functools.partial(_proj_body_first),
            grid=grid,
            in_specs=in_specs,
            out_specs=out_spec,
            out_shape=out_shape,
        )(emb, pje, pjo, emb3, p3e, p3o)
    return pl.pallas_call(
        _proj_body,
        grid=grid,
        in_specs=in_specs + [pl.BlockSpec(memory_space=pltpu.MemorySpace.HBM)],
        out_specs=out_spec,
        out_shape=out_shape,
        input_output_aliases={6: 0},
    )(emb, pje, pjo, emb3, p3e, p3o, unified)


def _proj_body_first(emb_ref, pje_ref, pjo_ref, e3_ref, p3e_ref, p3o_ref,
                     out_ref):
    _proj_body(emb_ref, pje_ref, pjo_ref, e3_ref, p3e_ref, p3o_ref,
               None, out_ref)


def _gather_body(tab_hbm, idx_hbm, out_hbm, idx_v, aidx_v, rows_v, fbuf_v,
                 *sems):
    gsem = sems[:_NBUF]
    wsem = sems[_NBUF:]
    wid = lax.axis_index("s") * _NC + lax.axis_index("c")
    base = wid * _BPW

    pltpu.sync_copy(idx_hbm.at[pl.ds(base, _BPW)], idx_v)

    def prep(i, carry):
        u = idx_v[pl.ds(i * 16, 16)]
        aidx_v[pl.ds(i * 16, 16)] = jnp.where(u >= _HALF, u - _HALF, u)
        return carry

    lax.fori_loop(0, _BPW // 16, prep, 0)

    def g_copy(c, b):
        return pltpu.make_async_copy(
            tab_hbm.at[aidx_v.at[pl.ds(c * _CH, _CH)]], rows_v.at[b], gsem[b]
        )

    def w_copy(c, b):
        return pltpu.make_async_copy(
            fbuf_v.at[b],
            out_hbm.at[pl.ds((base + c * _CH) * _D, _CH * _D)],
            wsem[b],
        )

    def convert(b, c):
        rows = rows_v.at[b]
        fb = fbuf_v.at[b]

        def one_group(tb, carry):
            uv = idx_v[pl.ds(c * _CH + tb * 16, 16)]
            for k in range(16):
                t = tb * 16 + k
                hoff = (uv[k] >= _HALF).astype(jnp.int32) * 64
                for j in range(4):
                    w = rows[t, pl.ds(hoff + j * 16, 16)]
                    lo = jax.lax.bitcast_convert_type(w << 16, jnp.float32)
                    hi = jax.lax.bitcast_convert_type(
                        w & jnp.int32(-65536), jnp.float32)
                    fb[pl.ds(t * _D + j * 32, 16)] = lo
                    fb[pl.ds(t * _D + j * 32 + 16, 16)] = hi
            return carry

        lax.fori_loop(0, _CH // 16, one_group, 0)

    for b in range(_NBUF):
        g_copy(b, b).start()

    for b in range(_NBUF):
        g_copy(b, b).wait()
        convert(b, b)
        w_copy(b, b).start()
        g_copy(b + _NBUF, b).start()

    def round_body(r, carry):
        for b in range(_NBUF):
            c = r * _NBUF + b
            g_copy(c, b).wait()
            w_copy(c - _NBUF, b).wait()
            convert(b, c)
            w_copy(c, b).start()
            g_copy(c + _NBUF, b).start()
        return carry

    lax.fori_loop(1, _NROUND - 1, round_body, 0)

    for b in range(_NBUF):
        c = (_NROUND - 1) * _NBUF + b
        g_copy(c, b).wait()
        w_copy(c - _NBUF, b).wait()
        convert(b, c)
        w_copy(c, b).start()
    for b in range(_NBUF):
        c = (_NROUND - 1) * _NBUF + b
        w_copy(c, b).wait()


def _sc_gather(unified, idx):
    mesh = plsc.VectorSubcoreMesh(
        core_axis_name="c", subcore_axis_name="s",
        num_cores=_NC, num_subcores=_NS,
    )
    scratch = [
        pltpu.VMEM((_BPW,), jnp.int32),
        pltpu.VMEM((_BPW,), jnp.int32),
        pltpu.VMEM((_NBUF, _CH, _D), jnp.int32),
        pltpu.VMEM((_NBUF, _CH * _D), jnp.float32),
    ] + [pltpu.SemaphoreType.DMA] * (2 * _NBUF)
    run = pl.kernel(
        _gather_body,
        out_type=jax.ShapeDtypeStruct((_B_TOTAL * _D,), jnp.float32),
        mesh=mesh,
        scratch_types=scratch,
    )
    return run(unified, idx)


def kernel(inp, emb0, emb1, emb2, emb3, proj0, proj1, proj2, proj3):
    idx = inp.reshape(-1).astype(jnp.int32)
    pjts = [p.T for p in (proj0, proj1, proj2, proj3)]
    cole = jnp.array([32 * j + k for j in range(4) for k in range(16)])
    colo = cole + 16
    pje = [p[:, cole] for p in pjts]
    pjo = [p[:, colo] for p in pjts]

    unified = None
    for c, emb in ((0, emb0), (1, emb1), (2, emb2)):
        l = _CUT[c]
        unified = _project_cluster(
            emb, pje[c], pjo[c], emb3, pje[3], pjo[3], unified, l
        )
    out = _sc_gather(unified, idx)
    return out.reshape(inp.shape + (_D,))


# speedup vs baseline: 1.0021x; 1.0021x over previous
import functools

import jax
import jax.numpy as jnp
from jax import lax
from jax.experimental import pallas as pl
from jax.experimental.pallas import tpu as pltpu
from jax.experimental.pallas import tpu_sc as plsc

_N_TOKEN = 1000000
_D = 128
_CUT = (0, 20000, 100000, 500000, 1000000)
_HALF = 500000
_SCALE = float(_D) ** 0.5

_BLK = 4000

_NC = 2
_NS = 16
_NW = _NC * _NS
_B_TOTAL = 4096 * 200
_BPW = _B_TOTAL // _NW
_CH = 128
_NCHUNK = _BPW // _CH
_NBUF = 2
_NROUND = _NCHUNK // _NBUF


def _pack_bf16_pair(dot_e, dot_o):
    e16 = jax.lax.bitcast_convert_type(dot_e.astype(jnp.bfloat16), jnp.int16)
    o16 = jax.lax.bitcast_convert_type(dot_o.astype(jnp.bfloat16), jnp.int16)
    return (o16.astype(jnp.int32) << 16) | (e16.astype(jnp.int32) & 0xFFFF)


def _proj_body(emb_ref, pje_ref, pjo_ref, e3_ref, p3e_ref, p3o_ref,
               u_ref, out_ref):
    del u_ref
    lo = _pack_bf16_pair(
        jnp.dot(emb_ref[...], pje_ref[...],
                preferred_element_type=jnp.float32) * _SCALE,
        jnp.dot(emb_ref[...], pjo_ref[...],
                preferred_element_type=jnp.float32) * _SCALE,
    )
    hi = _pack_bf16_pair(
        jnp.dot(e3_ref[...], p3e_ref[...],
                preferred_element_type=jnp.float32) * _SCALE,
        jnp.dot(e3_ref[...], p3o_ref[...],
                preferred_element_type=jnp.float32) * _SCALE,
    )
    out_ref[...] = jnp.concatenate([lo, hi], axis=1)


def _project_cluster(emb, pje, pjo, emb3, p3e, p3o, unified, row_off):
    rows, d = emb.shape
    grid = (rows // _BLK,)
    off_blocks = row_off // _BLK
    in_specs = [
        pl.BlockSpec((_BLK, d), lambda b: (b, 0)),
        pl.BlockSpec((d, 64), lambda b: (0, 0)),
        pl.BlockSpec((d, 64), lambda b: (0, 0)),
        pl.BlockSpec((_BLK, 2), lambda b, _o=off_blocks: (b + _o, 0)),
        pl.BlockSpec((2, 64), lambda b: (0, 0)),
        pl.BlockSpec((2, 64), lambda b: (0, 0)),
    ]
    out_spec = pl.BlockSpec((_BLK, _D), lambda b, _o=off_blocks: (b + _o, 0))
    out_shape = jax.ShapeDtypeStruct((_HALF, _D), jnp.int32)
    if unified is None:
        return pl.pallas_call(
            functools.partial(_proj_body_first),
            grid=grid,
            in_specs=in_specs,
            out_specs=out_spec,
            out_shape=out_shape,
        )(emb, pje, pjo, emb3, p3e, p3o)
    return pl.pallas_call(
        _proj_body,
        grid=grid,
        in_specs=in_specs + [pl.BlockSpec(memory_space=pltpu.MemorySpace.HBM)],
        out_specs=out_spec,
        out_shape=out_shape,
        input_output_aliases={6: 0},
    )(emb, pje, pjo, emb3, p3e, p3o, unified)


def _proj_body_first(emb_ref, pje_ref, pjo_ref, e3_ref, p3e_ref, p3o_ref,
                     out_ref):
    _proj_body(emb_ref, pje_ref, pjo_ref, e3_ref, p3e_ref, p3o_ref,
               None, out_ref)


def _gather_body(tab_hbm, idx_hbm, out_hbm, idx_v, aidx_v, rows_v, fbuf_v,
                 *sems):
    gsem = sems[:_NBUF]
    wsem = sems[_NBUF:]
    wid = lax.axis_index("s") * _NC + lax.axis_index("c")
    base = wid * _BPW

    pltpu.sync_copy(idx_hbm.at[pl.ds(base, _BPW)], idx_v)

    def prep(i, carry):
        u = idx_v[pl.ds(i * 16, 16)]
        aidx_v[pl.ds(i * 16, 16)] = jnp.where(u >= _HALF, u - _HALF, u)
        return carry

    lax.fori_loop(0, _BPW // 16, prep, 0)

    def g_copy(c, b):
        return pltpu.make_async_copy(
            tab_hbm.at[aidx_v.at[pl.ds(c * _CH, _CH)]], rows_v.at[b], gsem[b]
        )

    def w_copy(c, b):
        return pltpu.make_async_copy(
            fbuf_v.at[b],
            out_hbm.at[pl.ds((base + c * _CH) * _D, _CH * _D)],
            wsem[b],
        )

    def convert(b, c):
        rows = rows_v.at[b]
        fb = fbuf_v.at[b]

        def one_group(tb, carry):
            uv = idx_v[pl.ds(c * _CH + tb * 16, 16)]
            for k in range(16):
                t = tb * 16 + k
                hoff = (uv[k] >= _HALF).astype(jnp.int32) * 64
                for j in range(4):
                    w = rows[t, pl.ds(hoff + j * 16, 16)]
                    lo = jax.lax.bitcast_convert_type(w << 16, jnp.float32)
                    hi = jax.lax.bitcast_convert_type(
                        w & jnp.int32(-65536), jnp.float32)
                    fb[pl.ds(t * _D + j * 32, 16)] = lo
                    fb[pl.ds(t * _D + j * 32 + 16, 16)] = hi
            return carry

        lax.fori_loop(0, _CH // 16, one_group, 0)

    for b in range(_NBUF):
        g_copy(b, b).start()

    for b in range(_NBUF):
        g_copy(b, b).wait()
        convert(b, b)
        w_copy(b, b).start()
        g_copy(b + _NBUF, b).start()

    def round_body(r, carry):
        for b in range(_NBUF):
            c = r * _NBUF + b
            g_copy(c, b).wait()
            w_copy(c - _NBUF, b).wait()
            convert(b, c)
            w_copy(c, b).start()
            g_copy(c + _NBUF, b).start()
        return carry

    lax.fori_loop(1, _NROUND - 1, round_body, 0)

    for b in range(_NBUF):
        c = (_NROUND - 1) * _NBUF + b
        g_copy(c, b).wait()
        w_copy(c - _NBUF, b).wait()
        convert(b, c)
        w_copy(c, b).start()
    for b in range(_NBUF):
        c = (_NROUND - 1) * _NBUF + b
        w_copy(c, b).wait()


def _sc_gather(unified, idx):
    mesh = plsc.VectorSubcoreMesh(
        core_axis_name="c", subcore_axis_name="s",
        num_cores=_NC, num_subcores=_NS,
    )
    scratch = [
        pltpu.VMEM((_BPW,), jnp.int32),
        pltpu.VMEM((_BPW,), jnp.int32),
        pltpu.VMEM((_NBUF, _CH, _D), jnp.int32),
        pltpu.VMEM((_NBUF, _CH * _D), jnp.float32),
    ] + [pltpu.SemaphoreType.DMA] * (2 * _NBUF)
    run = pl.kernel(
        _gather_body,
        out_type=jax.ShapeDtypeStruct((_B_TOTAL * _D,), jnp.float32),
        mesh=mesh,
        scratch_types=scratch,
    )
    return run(unified, idx)


def kernel(inp, emb0, emb1, emb2, emb3, proj0, proj1, proj2, proj3):
    idx = inp.reshape(-1).astype(jnp.int32)
    pjts = [p.T for p in (proj0, proj1, proj2, proj3)]
    cole = jnp.array([32 * j + k for j in range(4) for k in range(16)])
    colo = cole + 16
    pje = [p[:, cole] for p in pjts]
    pjo = [p[:, colo] for p in pjts]

    unified = None
    for c, emb in ((0, emb0), (1, emb1), (2, emb2)):
        l = _CUT[c]
        unified = _project_cluster(
            emb, pje[c], pjo[c], emb3, pje[3], pjo[3], unified, l
        )
    out = _sc_gather(unified, idx)
    return out.reshape(inp.shape + (_D,))


# speedup vs baseline: 1.1183x; 1.1160x over previous
import functools

import jax
import jax.numpy as jnp
from jax import lax
from jax.experimental import pallas as pl
from jax.experimental.pallas import tpu as pltpu
from jax.experimental.pallas import tpu_sc as plsc

_N_TOKEN = 1000000
_D = 128
_CUT = (0, 20000, 100000, 500000, 1000000)
_HALF = 500000
_SCALE = float(_D) ** 0.5

_BLK = 4000

_NC = 2
_NS = 16
_NW = _NC * _NS
_B_TOTAL = 4096 * 200
_BPW = _B_TOTAL // _NW
_CH = 128
_NCHUNK = _BPW // _CH
_NBUF = 2
_NROUND = _NCHUNK // _NBUF


def _pack_bf16_pair(dot_e, dot_o):
    e16 = jax.lax.bitcast_convert_type(dot_e.astype(jnp.bfloat16), jnp.int16)
    o16 = jax.lax.bitcast_convert_type(dot_o.astype(jnp.bfloat16), jnp.int16)
    return (o16.astype(jnp.int32) << 16) | (e16.astype(jnp.int32) & 0xFFFF)


def _proj_body(emb_ref, pje_ref, pjo_ref, e3_ref, p3e_ref, p3o_ref,
               u_ref, out_ref):
    del u_ref
    lo = _pack_bf16_pair(
        jnp.dot(emb_ref[...], pje_ref[...],
                preferred_element_type=jnp.float32) * _SCALE,
        jnp.dot(emb_ref[...], pjo_ref[...],
                preferred_element_type=jnp.float32) * _SCALE,
    )
    hi = _pack_bf16_pair(
        jnp.dot(e3_ref[...], p3e_ref[...],
                preferred_element_type=jnp.float32) * _SCALE,
        jnp.dot(e3_ref[...], p3o_ref[...],
                preferred_element_type=jnp.float32) * _SCALE,
    )
    out_ref[...] = jnp.concatenate([lo, hi], axis=1)


def _project_cluster(emb, pje, pjo, emb3, p3e, p3o, unified, row_off):
    rows, d = emb.shape
    grid = (rows // _BLK,)
    off_blocks = row_off // _BLK
    in_specs = [
        pl.BlockSpec((_BLK, d), lambda b: (b, 0)),
        pl.BlockSpec((d, 64), lambda b: (0, 0)),
        pl.BlockSpec((d, 64), lambda b: (0, 0)),
        pl.BlockSpec((_BLK, 2), lambda b, _o=off_blocks: (b + _o, 0)),
        pl.BlockSpec((2, 64), lambda b: (0, 0)),
        pl.BlockSpec((2, 64), lambda b: (0, 0)),
    ]
    out_spec = pl.BlockSpec((_BLK, _D), lambda b, _o=off_blocks: (b + _o, 0))
    out_shape = jax.ShapeDtypeStruct((_HALF, _D), jnp.int32)
    if unified is None:
        return pl.pallas_call(
            functools.partial(_proj_body_first),
            grid=grid,
            in_specs=in_specs,
            out_specs=out_spec,
            out_shape=out_shape,
        )(emb, pje, pjo, emb3, p3e, p3o)
    return pl.pallas_call(
        _proj_body,
        grid=grid,
        in_specs=in_specs + [pl.BlockSpec(memory_space=pltpu.MemorySpace.HBM)],
        out_specs=out_spec,
        out_shape=out_shape,
        input_output_aliases={6: 0},
    )(emb, pje, pjo, emb3, p3e, p3o, unified)


def _proj_body_first(emb_ref, pje_ref, pjo_ref, e3_ref, p3e_ref, p3o_ref,
                     out_ref):
    _proj_body(emb_ref, pje_ref, pjo_ref, e3_ref, p3e_ref, p3o_ref,
               None, out_ref)


def _gather_body(tab_hbm, idx_hbm, out_hbm, idx_v, aidx_v, rows_v, fbuf_v,
                 *sems):
    gsem = sems[:_NBUF]
    wsem = sems[_NBUF:]
    wid = lax.axis_index("s") * _NC + lax.axis_index("c")
    base = wid * _BPW

    pltpu.sync_copy(idx_hbm.at[pl.ds(base, _BPW)], idx_v)

    def prep(i, carry):
        u = idx_v[pl.ds(i * 16, 16)]
        aidx_v[pl.ds(i * 16, 16)] = jnp.where(u >= _HALF, u - _HALF, u)
        return carry

    lax.fori_loop(0, _BPW // 16, prep, 0)

    def g_copy(c, b):
        return pltpu.make_async_copy(
            tab_hbm.at[aidx_v.at[pl.ds(c * _CH, _CH)]], rows_v.at[b], gsem[b]
        )

    def w_copy(c, b):
        return pltpu.make_async_copy(
            fbuf_v.at[b],
            out_hbm.at[pl.ds((base + c * _CH) * _D, _CH * _D)],
            wsem[b],
        )

    def convert(b, c):
        rows = rows_v.at[b]
        fb = fbuf_v.at[b]

        @plsc.parallel_loop(0, _CH // 16, unroll=2)
        def one_group(tb):
            uv = idx_v[pl.ds(c * _CH + tb * 16, 16)]
            for k in range(16):
                t = tb * 16 + k
                hoff = (uv[k] >= _HALF).astype(jnp.int32) * 64
                for j in range(4):
                    w = rows[t, pl.ds(hoff + j * 16, 16)]
                    lo = jax.lax.bitcast_convert_type(w << 16, jnp.float32)
                    hi = jax.lax.bitcast_convert_type(
                        w & jnp.int32(-65536), jnp.float32)
                    fb[pl.ds(t * _D + j * 32, 16)] = lo
                    fb[pl.ds(t * _D + j * 32 + 16, 16)] = hi

    for b in range(_NBUF):
        g_copy(b, b).start()

    for b in range(_NBUF):
        g_copy(b, b).wait()
        convert(b, b)
        w_copy(b, b).start()
        g_copy(b + _NBUF, b).start()

    def round_body(r, carry):
        for b in range(_NBUF):
            c = r * _NBUF + b
            g_copy(c, b).wait()
            w_copy(c - _NBUF, b).wait()
            convert(b, c)
            w_copy(c, b).start()
            g_copy(c + _NBUF, b).start()
        return carry

    lax.fori_loop(1, _NROUND - 1, round_body, 0)

    for b in range(_NBUF):
        c = (_NROUND - 1) * _NBUF + b
        g_copy(c, b).wait()
        w_copy(c - _NBUF, b).wait()
        convert(b, c)
        w_copy(c, b).start()
    for b in range(_NBUF):
        c = (_NROUND - 1) * _NBUF + b
        w_copy(c, b).wait()


def _sc_gather(unified, idx):
    mesh = plsc.VectorSubcoreMesh(
        core_axis_name="c", subcore_axis_name="s",
        num_cores=_NC, num_subcores=_NS,
    )
    scratch = [
        pltpu.VMEM((_BPW,), jnp.int32),
        pltpu.VMEM((_BPW,), jnp.int32),
        pltpu.VMEM((_NBUF, _CH, _D), jnp.int32),
        pltpu.VMEM((_NBUF, _CH * _D), jnp.float32),
    ] + [pltpu.SemaphoreType.DMA] * (2 * _NBUF)
    run = pl.kernel(
        _gather_body,
        out_type=jax.ShapeDtypeStruct((_B_TOTAL * _D,), jnp.float32),
        mesh=mesh,
        scratch_types=scratch,
    )
    return run(unified, idx)


def kernel(inp, emb0, emb1, emb2, emb3, proj0, proj1, proj2, proj3):
    idx = inp.reshape(-1).astype(jnp.int32)
    pjts = [p.T for p in (proj0, proj1, proj2, proj3)]
    cole = jnp.array([32 * j + k for j in range(4) for k in range(16)])
    colo = cole + 16
    pje = [p[:, cole] for p in pjts]
    pjo = [p[:, colo] for p in pjts]

    unified = None
    for c, emb in ((0, emb0), (1, emb1), (2, emb2)):
        l = _CUT[c]
        unified = _project_cluster(
            emb, pje[c], pjo[c], emb3, pje[3], pjo[3], unified, l
        )
    out = _sc_gather(unified, idx)
    return out.reshape(inp.shape + (_D,))


# speedup vs baseline: 1.1279x; 1.0086x over previous
import functools

import jax
import jax.numpy as jnp
from jax import lax
from jax.experimental import pallas as pl
from jax.experimental.pallas import tpu as pltpu
from jax.experimental.pallas import tpu_sc as plsc

_N_TOKEN = 1000000
_D = 128
_CUT = (0, 20000, 100000, 500000, 1000000)
_HALF = 500000
_SCALE = float(_D) ** 0.5

_BLK = 4000

_NC = 2
_NS = 16
_NW = _NC * _NS
_B_TOTAL = 4096 * 200
_BPW = _B_TOTAL // _NW
_CH = 128
_NCHUNK = _BPW // _CH
_NBUF = 2
_NROUND = _NCHUNK // _NBUF


def _pack_bf16_pair(dot_e, dot_o):
    e16 = jax.lax.bitcast_convert_type(dot_e.astype(jnp.bfloat16), jnp.int16)
    o16 = jax.lax.bitcast_convert_type(dot_o.astype(jnp.bfloat16), jnp.int16)
    return (o16.astype(jnp.int32) << 16) | (e16.astype(jnp.int32) & 0xFFFF)


def _proj_body(emb_ref, pje_ref, pjo_ref, e3_ref, p3e_ref, p3o_ref,
               u_ref, out_ref):
    del u_ref
    lo = _pack_bf16_pair(
        jnp.dot(emb_ref[...], pje_ref[...],
                preferred_element_type=jnp.float32) * _SCALE,
        jnp.dot(emb_ref[...], pjo_ref[...],
                preferred_element_type=jnp.float32) * _SCALE,
    )
    hi = _pack_bf16_pair(
        jnp.dot(e3_ref[...], p3e_ref[...],
                preferred_element_type=jnp.float32) * _SCALE,
        jnp.dot(e3_ref[...], p3o_ref[...],
                preferred_element_type=jnp.float32) * _SCALE,
    )
    out_ref[...] = jnp.concatenate([lo, hi], axis=1)


def _project_cluster(emb, pje, pjo, emb3, p3e, p3o, unified, row_off):
    rows, d = emb.shape
    grid = (rows // _BLK,)
    off_blocks = row_off // _BLK
    in_specs = [
        pl.BlockSpec((_BLK, d), lambda b: (b, 0)),
        pl.BlockSpec((d, 64), lambda b: (0, 0)),
        pl.BlockSpec((d, 64), lambda b: (0, 0)),
        pl.BlockSpec((_BLK, 2), lambda b, _o=off_blocks: (b + _o, 0)),
        pl.BlockSpec((2, 64), lambda b: (0, 0)),
        pl.BlockSpec((2, 64), lambda b: (0, 0)),
    ]
    out_spec = pl.BlockSpec((_BLK, _D), lambda b, _o=off_blocks: (b + _o, 0))
    out_shape = jax.ShapeDtypeStruct((_HALF, _D), jnp.int32)
    if unified is None:
        return pl.pallas_call(
            functools.partial(_proj_body_first),
            grid=grid,
            in_specs=in_specs,
            out_specs=out_spec,
            out_shape=out_shape,
        )(emb, pje, pjo, emb3, p3e, p3o)
    return pl.pallas_call(
        _proj_body,
        grid=grid,
        in_specs=in_specs + [pl.BlockSpec(memory_space=pltpu.MemorySpace.HBM)],
        out_specs=out_spec,
        out_shape=out_shape,
        input_output_aliases={6: 0},
    )(emb, pje, pjo, emb3, p3e, p3o, unified)


def _proj_body_first(emb_ref, pje_ref, pjo_ref, e3_ref, p3e_ref, p3o_ref,
                     out_ref):
    _proj_body(emb_ref, pje_ref, pjo_ref, e3_ref, p3e_ref, p3o_ref,
               None, out_ref)


def _gather_body(tab_hbm, idx_hbm, out_hbm, idx_v, aidx_v, rows_v, fbuf_v,
                 *sems):
    gsem = sems[:_NBUF]
    wsem = sems[_NBUF:]
    wid = lax.axis_index("s") * _NC + lax.axis_index("c")
    base = wid * _BPW

    pltpu.sync_copy(idx_hbm.at[pl.ds(base, _BPW)], idx_v)

    def prep(i, carry):
        u = idx_v[pl.ds(i * 16, 16)]
        aidx_v[pl.ds(i * 16, 16)] = jnp.where(u >= _HALF, u - _HALF, u)
        return carry

    lax.fori_loop(0, _BPW // 16, prep, 0)

    def g_copy(c, b):
        return pltpu.make_async_copy(
            tab_hbm.at[aidx_v.at[pl.ds(c * _CH, _CH)]], rows_v.at[b], gsem[b]
        )

    def w_copy(c, b):
        return pltpu.make_async_copy(
            fbuf_v.at[b],
            out_hbm.at[pl.ds((base + c * _CH) * _D, _CH * _D)],
            wsem[b],
        )

    def convert(b, c):
        rows = rows_v.at[b]
        fb = fbuf_v.at[b]

        @plsc.parallel_loop(0, _CH // 16, unroll=4)
        def one_group(tb):
            uv = idx_v[pl.ds(c * _CH + tb * 16, 16)]
            for k in range(16):
                t = tb * 16 + k
                hoff = (uv[k] >= _HALF).astype(jnp.int32) * 64
                for j in range(4):
                    w = rows[t, pl.ds(hoff + j * 16, 16)]
                    lo = jax.lax.bitcast_convert_type(w << 16, jnp.float32)
                    hi = jax.lax.bitcast_convert_type(
                        w & jnp.int32(-65536), jnp.float32)
                    fb[pl.ds(t * _D + j * 32, 16)] = lo
                    fb[pl.ds(t * _D + j * 32 + 16, 16)] = hi

    for b in range(_NBUF):
        g_copy(b, b).start()

    for b in range(_NBUF):
        g_copy(b, b).wait()
        convert(b, b)
        w_copy(b, b).start()
        g_copy(b + _NBUF, b).start()

    def round_body(r, carry):
        for b in range(_NBUF):
            c = r * _NBUF + b
            g_copy(c, b).wait()
            w_copy(c - _NBUF, b).wait()
            convert(b, c)
            w_copy(c, b).start()
            g_copy(c + _NBUF, b).start()
        return carry

    lax.fori_loop(1, _NROUND - 1, round_body, 0)

    for b in range(_NBUF):
        c = (_NROUND - 1) * _NBUF + b
        g_copy(c, b).wait()
        w_copy(c - _NBUF, b).wait()
        convert(b, c)
        w_copy(c, b).start()
    for b in range(_NBUF):
        c = (_NROUND - 1) * _NBUF + b
        w_copy(c, b).wait()


def _sc_gather(unified, idx):
    mesh = plsc.VectorSubcoreMesh(
        core_axis_name="c", subcore_axis_name="s",
        num_cores=_NC, num_subcores=_NS,
    )
    scratch = [
        pltpu.VMEM((_BPW,), jnp.int32),
        pltpu.VMEM((_BPW,), jnp.int32),
        pltpu.VMEM((_NBUF, _CH, _D), jnp.int32),
        pltpu.VMEM((_NBUF, _CH * _D), jnp.float32),
    ] + [pltpu.SemaphoreType.DMA] * (2 * _NBUF)
    run = pl.kernel(
        _gather_body,
        out_type=jax.ShapeDtypeStruct((_B_TOTAL * _D,), jnp.float32),
        mesh=mesh,
        scratch_types=scratch,
    )
    return run(unified, idx)


def kernel(inp, emb0, emb1, emb2, emb3, proj0, proj1, proj2, proj3):
    idx = inp.reshape(-1).astype(jnp.int32)
    pjts = [p.T for p in (proj0, proj1, proj2, proj3)]
    cole = jnp.array([32 * j + k for j in range(4) for k in range(16)])
    colo = cole + 16
    pje = [p[:, cole] for p in pjts]
    pjo = [p[:, colo] for p in pjts]

    unified = None
    for c, emb in ((0, emb0), (1, emb1), (2, emb2)):
        l = _CUT[c]
        unified = _project_cluster(
            emb, pje[c], pjo[c], emb3, pje[3], pjo[3], unified, l
        )
    out = _sc_gather(unified, idx)
    return out.reshape(inp.shape + (_D,))


# speedup vs baseline: 1.2856x; 1.1398x over previous
import jax
import jax.numpy as jnp
from jax import lax
from jax.experimental import pallas as pl
from jax.experimental.pallas import tpu as pltpu
from jax.experimental.pallas import tpu_sc as plsc

_N_TOKEN = 1000000
_D = 128
_CUT = (0, 20000, 100000, 500000, 1000000)
_SCALE = float(_D) ** 0.5

_BLK = 4000

_NC = 2
_NS = 16
_NW = _NC * _NS
_B_TOTAL = 4096 * 200
_BPW = _B_TOTAL // _NW
_CH = 64
_NCHUNK = _BPW // _CH
_NSLOT = 8
_LA = 4
_NROUND = _NCHUNK // _NSLOT


def _proj_body(emb_ref, pjt_ref, u_ref, out_ref):
    del u_ref
    out_ref[...] = (
        jnp.dot(emb_ref[...], pjt_ref[...], preferred_element_type=jnp.float32)
        * _SCALE
    )


def _project_cluster(emb, pjt, unified, row_off):
    rows, d = emb.shape
    grid = (rows // _BLK,)
    off_blocks = row_off // _BLK
    return pl.pallas_call(
        _proj_body,
        grid=grid,
        in_specs=[
            pl.BlockSpec((_BLK, d), lambda b: (b, 0)),
            pl.BlockSpec((d, _D), lambda b: (0, 0)),
            pl.BlockSpec(memory_space=pltpu.MemorySpace.HBM),
        ],
        out_specs=pl.BlockSpec((_BLK, _D), lambda b, _o=off_blocks: (b + _o, 0)),
        out_shape=jax.ShapeDtypeStruct((_N_TOKEN, _D), jnp.float32),
        input_output_aliases={2: 0},
    )(emb, pjt, unified)


def _proj_body_first(emb_ref, pjt_ref, out_ref):
    out_ref[...] = (
        jnp.dot(emb_ref[...], pjt_ref[...], preferred_element_type=jnp.float32)
        * _SCALE
    )


def _project_first(emb, pjt):
    rows, d = emb.shape
    return pl.pallas_call(
        _proj_body_first,
        grid=(rows // _BLK,),
        in_specs=[
            pl.BlockSpec((_BLK, d), lambda b: (b, 0)),
            pl.BlockSpec((d, _D), lambda b: (0, 0)),
        ],
        out_specs=pl.BlockSpec((_BLK, _D), lambda b: (b, 0)),
        out_shape=jax.ShapeDtypeStruct((_N_TOKEN, _D), jnp.float32),
    )(emb, pjt)


def _gather_body(tab_hbm, idx_hbm, out_hbm, idx_v, rows_v, *sems):
    gsem = sems[:_NSLOT]
    wsem = sems[_NSLOT:]
    wid = lax.axis_index("s") * _NC + lax.axis_index("c")
    base = wid * _BPW

    pltpu.sync_copy(idx_hbm.at[pl.ds(base, _BPW)], idx_v)

    def g_copy(c, s):
        return pltpu.make_async_copy(
            tab_hbm.at[idx_v.at[pl.ds(c * _CH, _CH)]], rows_v.at[s], gsem[s]
        )

    def w_copy(c, s):
        return pltpu.make_async_copy(
            rows_v.at[s], out_hbm.at[pl.ds(base + c * _CH, _CH)], wsem[s]
        )

    for b in range(_LA):
        g_copy(b, b).start()

    for b in range(_NSLOT):
        p = b + _LA
        if p >= _NSLOT:
            w_copy(p - _NSLOT, p % _NSLOT).wait()
        g_copy(p, p % _NSLOT).start()
        g_copy(b, b).wait()
        w_copy(b, b).start()

    def round_body(r, carry):
        for b in range(_NSLOT):
            c = r * _NSLOT + b
            sp = (b + _LA) % _NSLOT
            w_copy(c + _LA - _NSLOT, sp).wait()
            g_copy(c + _LA, sp).start()
            g_copy(c, b).wait()
            w_copy(c, b).start()
        return carry

    lax.fori_loop(1, _NROUND - 1, round_body, 0)

    for b in range(_NSLOT):
        c = (_NROUND - 1) * _NSLOT + b
        if c + _LA < _NCHUNK:
            sp = (b + _LA) % _NSLOT
            w_copy(c + _LA - _NSLOT, sp).wait()
            g_copy(c + _LA, sp).start()
        g_copy(c, b).wait()
        w_copy(c, b).start()

    for b in range(_NSLOT):
        c = (_NROUND - 1) * _NSLOT + b
        w_copy(c, b).wait()


def _sc_gather(unified, idx):
    mesh = plsc.VectorSubcoreMesh(
        core_axis_name="c", subcore_axis_name="s",
        num_cores=_NC, num_subcores=_NS,
    )
    scratch = [
        pltpu.VMEM((_BPW,), jnp.int32),
        pltpu.VMEM((_NSLOT, _CH, _D), jnp.float32),
    ] + [pltpu.SemaphoreType.DMA] * (2 * _NSLOT)
    run = pl.kernel(
        _gather_body,
        out_type=jax.ShapeDtypeStruct((_B_TOTAL, _D), jnp.float32),
        mesh=mesh,
        scratch_types=scratch,
    )
    return run(unified, idx)


def kernel(inp, emb0, emb1, emb2, emb3, proj0, proj1, proj2, proj3):
    idx = inp.reshape(-1).astype(jnp.int32)
    unified = _project_first(emb0, proj0.T)
    unified = _project_cluster(emb1, proj1.T, unified, _CUT[1])
    unified = _project_cluster(emb2, proj2.T, unified, _CUT[2])
    unified = _project_cluster(emb3, proj3.T, unified, _CUT[3])
    out = _sc_gather(unified, idx)
    return out.reshape(inp.shape + (_D,))
